# baseline (speedup 1.0000x reference)
import functools

import jax
import jax.numpy as jnp
from jax import lax
from jax.experimental import pallas as pl
from jax.experimental.pallas import tpu as pltpu
from jax.experimental.pallas import tpu_sc as plsc

B = 1024
D = 16
LANES = 128

_info = plsc.get_sparse_core_info()
_NC, _NS = _info.num_cores, _info.num_subcores
_NW = _NC * _NS
_BPW = B // _NW
_ROUND = 16

_sc_mesh = plsc.VectorSubcoreMesh(core_axis_name="c", subcore_axis_name="s")


@functools.partial(
    pl.kernel,
    mesh=_sc_mesh,
    compiler_params=pltpu.CompilerParams(needs_layout_passes=False),
    out_type=[
        jax.ShapeDtypeStruct((B, D), jnp.float32),
        jax.ShapeDtypeStruct((B, D), jnp.float32),
        jax.ShapeDtypeStruct((B,), jnp.float32),
        jax.ShapeDtypeStruct((B,), jnp.float32),
    ],
    scratch_types=[
        pltpu.VMEM((_BPW,), jnp.int32),
        pltpu.VMEM((_BPW,), jnp.int32),
        pltpu.VMEM((_ROUND, D, LANES), jnp.float32),
        pltpu.VMEM((_ROUND, D, LANES), jnp.float32),
        pltpu.VMEM((_BPW, D), jnp.float32),
        pltpu.VMEM((_BPW, D), jnp.float32),
        pltpu.VMEM((_BPW,), jnp.float32),
        pltpu.VMEM((_BPW,), jnp.float32),
        pltpu.SemaphoreType.DMA,
        pltpu.SemaphoreType.DMA,
    ],
)
def _sc_gather(uid_hbm, iid_hbm, ut_hbm, it_hbm, ub_hbm, ib_hbm,
               urep_hbm, irep_hbm, ubias_hbm, ibias_hbm,
               uidx_v, iidx_v, ustage, istage, urow_v, irow_v,
               ub_v, ib_v, sem, semb):
    wid = lax.axis_index("s") * _NC + lax.axis_index("c")
    base = wid * _BPW
    pltpu.sync_copy(uid_hbm.at[pl.ds(base, _BPW)], uidx_v)
    pltpu.sync_copy(iid_hbm.at[pl.ds(base, _BPW)], iidx_v)
    cub = pltpu.async_copy(ub_hbm.at[0].at[uidx_v], ub_v, semb)
    cib = pltpu.async_copy(ib_hbm.at[0].at[iidx_v], ib_v, semb)
    lane_iota = lax.iota(jnp.int32, 16)
    for rnd in range(_BPW // _ROUND):
        uvec = uidx_v[pl.ds(rnd * _ROUND, 16)]
        ivec = iidx_v[pl.ds(rnd * _ROUND, 16)]
        copies = []
        rms = []
        for j in range(_ROUND):
            ru = uvec[j]
            ri = ivec[j]
            rbu = pl.multiple_of((ru // LANES) * LANES, LANES)
            rbi = pl.multiple_of((ri // LANES) * LANES, LANES)
            rms.append((ru - rbu, ri - rbi))
            copies.append(pltpu.async_copy(
                ut_hbm.at[:, pl.ds(rbu, LANES)], ustage.at[j], sem))
            copies.append(pltpu.async_copy(
                it_hbm.at[:, pl.ds(rbi, LANES)], istage.at[j], sem))
        for c in copies:
            c.wait()
        for j in range(_ROUND):
            k = rnd * _ROUND + j
            rmu, rmi = rms[j]
            uvecd = plsc.load_gather(
                ustage.at[j], [lane_iota, lax.broadcast(rmu, (16,))])
            ivecd = plsc.load_gather(
                istage.at[j], [lane_iota, lax.broadcast(rmi, (16,))])
            urow_v[k] = uvecd
            irow_v[k] = ivecd
    cub.wait()
    cib.wait()
    pltpu.sync_copy(urow_v, urep_hbm.at[pl.ds(base, _BPW), :])
    pltpu.sync_copy(irow_v, irep_hbm.at[pl.ds(base, _BPW), :])
    pltpu.sync_copy(ub_v, ubias_hbm.at[pl.ds(base, _BPW)])
    pltpu.sync_copy(ib_v, ibias_hbm.at[pl.ds(base, _BPW)])


_G = 1
_BR = B // _G


def _tc_body(irep_ref, urep_ref, ub_ref, ib_ref, out_ref):
    scores = lax.dot_general(
        irep_ref[...], urep_ref[...],
        dimension_numbers=(((1,), (1,)), ((), ())),
        preferred_element_type=jnp.float32,
    ) + (ub_ref[...] + ib_ref[...])
    out_ref[...] = scores.reshape(_BR * 8, LANES)


_tc_matmul = pl.pallas_call(
    _tc_body,
    grid=(_G,),
    in_specs=[
        pl.BlockSpec((_BR, D), lambda i: (i, 0)),
        pl.BlockSpec((B, D), lambda i: (0, 0)),
        pl.BlockSpec((1, B), lambda i: (0, 0)),
        pl.BlockSpec((1, B), lambda i: (0, 0)),
    ],
    out_specs=pl.BlockSpec((_BR * 8, LANES), lambda i: (i, 0)),
    out_shape=jax.ShapeDtypeStruct((B * 8, LANES), jnp.float32),
)


def kernel(user_id, item_id, user_table, item_table, user_bias_table, item_bias_table):
    urep, irep, ubias, ibias = _sc_gather(
        user_id.astype(jnp.int32), item_id.astype(jnp.int32),
        user_table.T, item_table.T,
        user_bias_table.T, item_bias_table.T)
    out = _tc_matmul(irep, urep, ubias.reshape(1, B), ibias.reshape(1, B))
    return out.reshape(B, B, 1)


# speedup vs baseline: 1.0122x; 1.0122x over previous
import functools

import jax
import jax.numpy as jnp
from jax import lax
from jax.experimental import pallas as pl
from jax.experimental.pallas import tpu as pltpu
from jax.experimental.pallas import tpu_sc as plsc

B = 1024
D = 16
LANES = 128

_info = plsc.get_sparse_core_info()
_NC, _NS = _info.num_cores, _info.num_subcores
_NW = _NC * _NS
_BPW = B // _NW
_ROUND = 16

_sc_mesh = plsc.VectorSubcoreMesh(core_axis_name="c", subcore_axis_name="s")


@functools.partial(
    pl.kernel,
    mesh=_sc_mesh,
    compiler_params=pltpu.CompilerParams(needs_layout_passes=False),
    out_type=[
        jax.ShapeDtypeStruct((B, D), jnp.float32),
        jax.ShapeDtypeStruct((B, D), jnp.float32),
        jax.ShapeDtypeStruct((B,), jnp.float32),
        jax.ShapeDtypeStruct((B,), jnp.float32),
    ],
    scratch_types=[
        pltpu.VMEM((_BPW,), jnp.int32),
        pltpu.VMEM((_BPW,), jnp.int32),
        pltpu.VMEM((_ROUND, D, LANES), jnp.float32),
        pltpu.VMEM((_ROUND, D, LANES), jnp.float32),
        pltpu.VMEM((_BPW, D), jnp.float32),
        pltpu.VMEM((_BPW, D), jnp.float32),
        pltpu.VMEM((_BPW,), jnp.float32),
        pltpu.VMEM((_BPW,), jnp.float32),
        pltpu.SemaphoreType.DMA,
        pltpu.SemaphoreType.DMA,
    ],
)
def _sc_gather(uid_hbm, iid_hbm, ut_hbm, it_hbm, ub_hbm, ib_hbm,
               urep_hbm, irep_hbm, ubias_hbm, ibias_hbm,
               uidx_v, iidx_v, ustage, istage, urow_v, irow_v,
               ub_v, ib_v, sem, semb):
    wid = lax.axis_index("s") * _NC + lax.axis_index("c")
    base = wid * _BPW
    pltpu.sync_copy(uid_hbm.at[pl.ds(base, _BPW)], uidx_v)
    pltpu.sync_copy(iid_hbm.at[pl.ds(base, _BPW)], iidx_v)
    cub = pltpu.async_copy(ub_hbm.at[0].at[uidx_v], ub_v, semb)
    cib = pltpu.async_copy(ib_hbm.at[0].at[iidx_v], ib_v, semb)
    lane_iota = lax.iota(jnp.int32, 16)
    for rnd in range(_BPW // _ROUND):
        uvec = uidx_v[pl.ds(rnd * _ROUND, 16)]
        ivec = iidx_v[pl.ds(rnd * _ROUND, 16)]
        copies = []
        rms = []
        for j in range(_ROUND):
            ru = uvec[j]
            ri = ivec[j]
            rbu = pl.multiple_of((ru // LANES) * LANES, LANES)
            rbi = pl.multiple_of((ri // LANES) * LANES, LANES)
            rms.append((ru - rbu, ri - rbi))
            copies.append(pltpu.async_copy(
                ut_hbm.at[:, pl.ds(rbu, LANES)], ustage.at[j], sem))
            copies.append(pltpu.async_copy(
                it_hbm.at[:, pl.ds(rbi, LANES)], istage.at[j], sem))
        for c in copies:
            c.wait()
        for j in range(_ROUND):
            k = rnd * _ROUND + j
            rmu, rmi = rms[j]
            uvecd = plsc.load_gather(
                ustage.at[j], [lane_iota, lax.broadcast(rmu, (16,))])
            ivecd = plsc.load_gather(
                istage.at[j], [lane_iota, lax.broadcast(rmi, (16,))])
            urow_v[k] = uvecd
            irow_v[k] = ivecd
    cub.wait()
    cib.wait()
    pltpu.sync_copy(urow_v, urep_hbm.at[pl.ds(base, _BPW), :])
    pltpu.sync_copy(irow_v, irep_hbm.at[pl.ds(base, _BPW), :])
    pltpu.sync_copy(ub_v, ubias_hbm.at[pl.ds(base, _BPW)])
    pltpu.sync_copy(ib_v, ibias_hbm.at[pl.ds(base, _BPW)])


_G = 2
_BR = B // _G


def _tc_body(irep_ref, urep_ref, ub_ref, ib_ref, out_ref):
    scores = lax.dot_general(
        irep_ref[...], urep_ref[...],
        dimension_numbers=(((1,), (1,)), ((), ())),
        preferred_element_type=jnp.float32,
    ) + (ub_ref[...] + ib_ref[...])
    out_ref[...] = scores.reshape(_BR * 8, LANES)


_tc_matmul = pl.pallas_call(
    _tc_body,
    grid=(_G,),
    in_specs=[
        pl.BlockSpec((_BR, D), lambda i: (i, 0)),
        pl.BlockSpec((B, D), lambda i: (0, 0)),
        pl.BlockSpec((1, B), lambda i: (0, 0)),
        pl.BlockSpec((1, B), lambda i: (0, 0)),
    ],
    out_specs=pl.BlockSpec((_BR * 8, LANES), lambda i: (i, 0)),
    out_shape=jax.ShapeDtypeStruct((B * 8, LANES), jnp.float32),
)


def kernel(user_id, item_id, user_table, item_table, user_bias_table, item_bias_table):
    urep, irep, ubias, ibias = _sc_gather(
        user_id.astype(jnp.int32), item_id.astype(jnp.int32),
        user_table.T, item_table.T,
        user_bias_table.T, item_bias_table.T)
    out = _tc_matmul(irep, urep, ubias.reshape(1, B), ibias.reshape(1, B))
    return out.reshape(B, B, 1)
